# baseline (speedup 1.0000x reference)
import jax
import jax.numpy as jnp
from jax import lax
from jax.experimental import pallas as pl
from jax.experimental.pallas import tpu as pltpu
from jax.experimental.pallas import tpu_sc as plsc

NC = 2
NS = 16
L = 16
BATCH = 4096
HIST = 50
DIM = 50
DIMP = 64
CLASS_NUM = 4

BAGS_PER_GATHER = 16
G_ROWS = BAGS_PER_GATHER * HIST
N_GATHERS = BATCH // NS // BAGS_PER_GATHER
NBUF = 2
GROUPS = N_GATHERS // NBUF


def _sc_body(xr_hbm, xs_hbm, etd_hbm, ewae_hbm, outr_hbm, outw_hbm,
             idx_v, g0, g1,
             maxacc, outv, redv, resv, shared,
             s0, s1):
    c = lax.axis_index("c")
    s = lax.axis_index("s")
    gbufs = (g0, g1)
    sems = (s0, s1)

    @pl.when(c == 0)
    def _():
        pltpu.sync_copy(xr_hbm.at[s], idx_v)

    @pl.when(c == 1)
    def _():
        pltpu.sync_copy(xs_hbm.at[s], idx_v)

    for q in range(4):
        maxacc[q, :] = jnp.full((L,), -jnp.inf, jnp.float32)

    def fire(row, b):
        @pl.when(c == 0)
        def _():
            pltpu.async_copy(etd_hbm.at[idx_v.at[row]], gbufs[b], sems[b])

        @pl.when(c == 1)
        def _():
            pltpu.async_copy(ewae_hbm.at[idx_v.at[row]], gbufs[b], sems[b])

    for b in range(NBUF):
        fire(b, b)

    def group(g, carry):
        for b in range(NBUF):
            row = g * NBUF + b
            pltpu.make_async_copy(
                etd_hbm.at[idx_v.at[row]], gbufs[b], sems[b]).wait()
            gb = gbufs[b]

            def bag_body(bag, bcarry):
                base = bag * HIST
                a0 = gb[base, pl.ds(0, L)]
                a1 = gb[base, pl.ds(16, L)]
                a2 = gb[base, pl.ds(32, L)]
                a3 = gb[base, pl.ds(40, L)]
                for r in range(1, HIST):
                    a0 = a0 + gb[base + r, pl.ds(0, L)]
                    a1 = a1 + gb[base + r, pl.ds(16, L)]
                    a2 = a2 + gb[base + r, pl.ds(32, L)]
                    a3 = a3 + gb[base + r, pl.ds(40, L)]
                for q, a in enumerate((a0, a1, a2, a3)):
                    lr = jnp.where(a >= 0.0, a, a * 0.01)
                    maxacc[q, :] = jnp.maximum(maxacc[q, :], lr)
                return bcarry

            lax.fori_loop(0, BAGS_PER_GATHER, bag_body, 0)

            @pl.when(row + NBUF < N_GATHERS)
            def _():
                fire(row + NBUF, b)
        return carry

    lax.fori_loop(0, GROUPS, group, 0)

    outv[pl.ds(48, L)] = jnp.zeros((L,), jnp.float32)
    outv[pl.ds(0, L)] = maxacc[0, :]
    outv[pl.ds(16, L)] = maxacc[1, :]
    outv[pl.ds(32, L)] = maxacc[2, :]
    outv[pl.ds(40, L)] = maxacc[3, :]
    pltpu.sync_copy(outv, shared.at[s])
    plsc.subcore_barrier()

    @pl.when(s == 0)
    def _():
        pltpu.sync_copy(shared, redv)
        m0 = redv[0, pl.ds(0, L)]
        m1 = redv[0, pl.ds(16, L)]
        m2 = redv[0, pl.ds(32, L)]
        m3 = redv[0, pl.ds(48, L)]
        for t in range(1, NS):
            m0 = jnp.maximum(m0, redv[t, pl.ds(0, L)])
            m1 = jnp.maximum(m1, redv[t, pl.ds(16, L)])
            m2 = jnp.maximum(m2, redv[t, pl.ds(32, L)])
            m3 = jnp.maximum(m3, redv[t, pl.ds(48, L)])
        resv[pl.ds(0, L)] = m0
        resv[pl.ds(16, L)] = m1
        resv[pl.ds(32, L)] = m2
        resv[pl.ds(48, L)] = m3

        @pl.when(c == 0)
        def _():
            pltpu.sync_copy(resv, outr_hbm)

        @pl.when(c == 1)
        def _():
            pltpu.sync_copy(resv, outw_hbm)


REPACK_R = 4000
REPACK_H = REPACK_R // 2
REPACK_G = 100000 // REPACK_R


def _repack_body(a_ref, b_ref, oa_ref, ob_ref):
    zp = ((0, 0), (0, DIMP - DIM))
    for src_ref, dst_ref in ((a_ref, oa_ref), (b_ref, ob_ref)):
        lo = jnp.pad(src_ref[0:REPACK_H, :], zp)
        hi = jnp.pad(src_ref[REPACK_H:REPACK_R, :], zp)
        dst_ref[...] = jnp.concatenate([lo, hi], axis=1)


def _repack(E_a, E_b):
    return pl.pallas_call(
        _repack_body,
        grid=(REPACK_G,),
        in_specs=[
            pl.BlockSpec((REPACK_R, DIM), lambda i: (i, 0)),
            pl.BlockSpec((REPACK_R, DIM), lambda i: (i, 0)),
        ],
        out_specs=[
            pl.BlockSpec((REPACK_H, 2 * DIMP), lambda i: (i, 0)),
            pl.BlockSpec((REPACK_H, 2 * DIMP), lambda i: (i, 0)),
        ],
        out_shape=[
            jax.ShapeDtypeStruct((50000, 2 * DIMP), jnp.float32),
            jax.ShapeDtypeStruct((50000, 2 * DIMP), jnp.float32),
        ],
    )(E_a, E_b)


def _head_body(pr_ref, pw_ref, w_ref, b_ref, y_ref, pred_ref, loss_ref):
    pr = pr_ref[0:1, 0:DIM]
    pw = pw_ref[0:1, 0:DIM]
    path = jnp.concatenate([pr, pw], axis=1)
    w = w_ref[...]
    logits = jnp.sum(w * path, axis=1, keepdims=True).T + b_ref[...]
    m = jnp.max(logits, axis=1, keepdims=True)
    e = jnp.exp(logits - m)
    p = e / jnp.sum(e, axis=1, keepdims=True)
    pred_ref[...] = p
    is_one = y_ref[...] == 1.0
    ii = lax.broadcasted_iota(jnp.int32, (1, CLASS_NUM), 1).astype(jnp.float32)
    cand = jnp.where(is_one, ii, jnp.float32(CLASS_NUM))
    idx_first = jnp.min(cand)
    label = jnp.where(idx_first < CLASS_NUM, idx_first, 0.0)
    sel = (ii == label).astype(jnp.float32)
    pm = jnp.max(p, axis=1, keepdims=True)
    lse = pm + jnp.log(jnp.sum(jnp.exp(p - pm), axis=1, keepdims=True))
    p_label = jnp.sum(p * sel, axis=1, keepdims=True)
    loss_ref[...] = lse - p_label


@jax.jit
def kernel(x_random, x_response, y, E_td, E_wae, w_cat, b_cat):
    def remap(x):
        b = x // REPACK_R
        r = x % REPACK_R
        return 2 * (REPACK_H * b + r % REPACK_H) + r // REPACK_H

    xr = remap(x_random).reshape(NS, N_GATHERS, G_ROWS)
    xs = remap(x_response).reshape(NS, N_GATHERS, G_ROWS)
    etd_p, ewae_p = _repack(E_td, E_wae)
    etd = etd_p.reshape(100000, DIMP)
    ewae = ewae_p.reshape(100000, DIMP)

    sc = pl.kernel(
        _sc_body,
        out_type=[
            jax.ShapeDtypeStruct((4 * L,), jnp.float32),
            jax.ShapeDtypeStruct((4 * L,), jnp.float32),
        ],
        mesh=plsc.VectorSubcoreMesh(core_axis_name="c", subcore_axis_name="s"),
        compiler_params=pltpu.CompilerParams(use_tc_tiling_on_sc=False),
        scratch_types=(
            [pltpu.VMEM((N_GATHERS, G_ROWS), jnp.int32)]
            + [pltpu.VMEM((G_ROWS, DIMP), jnp.float32)] * NBUF
            + [
                pltpu.VMEM((4, L), jnp.float32),
                pltpu.VMEM((4 * L,), jnp.float32),
                pltpu.VMEM((NS, 4 * L), jnp.float32),
                pltpu.VMEM((4 * L,), jnp.float32),
                pltpu.VMEM_SHARED((NS, 4 * L), jnp.float32),
            ]
            + [pltpu.SemaphoreType.DMA] * NBUF
        ),
    )
    path_r, path_w = sc(xr, xs, etd, ewae)

    pred2, loss2 = pl.pallas_call(
        _head_body,
        out_shape=[
            jax.ShapeDtypeStruct((1, CLASS_NUM), jnp.float32),
            jax.ShapeDtypeStruct((1, 1), jnp.float32),
        ],
    )(path_r.reshape(1, 4 * L), path_w.reshape(1, 4 * L),
      w_cat, b_cat.reshape(1, CLASS_NUM), y.reshape(1, CLASS_NUM))

    return (pred2[0], loss2[0, 0])


# speedup vs baseline: 1.4527x; 1.4527x over previous
import jax
import jax.numpy as jnp
from jax import lax
from jax.experimental import pallas as pl
from jax.experimental.pallas import tpu as pltpu
from jax.experimental.pallas import tpu_sc as plsc

NC = 2
NS = 16
L = 16
BATCH = 4096
HIST = 50
DIM = 50
DIMP = 64
CLASS_NUM = 4

BAGS_PER_GATHER = 16
G_ROWS = BAGS_PER_GATHER * HIST
N_GATHERS = BATCH // NS // BAGS_PER_GATHER
NBUF = 2
GROUPS = N_GATHERS // NBUF


def _sc_body(xr_hbm, xs_hbm, etd_hbm, ewae_hbm, outr_hbm, outw_hbm,
             idx_v, g0, g1,
             maxacc, outv, redv, resv, shared,
             s0, s1):
    c = lax.axis_index("c")
    s = lax.axis_index("s")
    gbufs = (g0, g1)
    sems = (s0, s1)

    @pl.when(c == 0)
    def _():
        pltpu.sync_copy(xr_hbm.at[s], idx_v)

    @pl.when(c == 1)
    def _():
        pltpu.sync_copy(xs_hbm.at[s], idx_v)

    for q in range(4):
        maxacc[q, :] = jnp.full((L,), -jnp.inf, jnp.float32)

    def fire(row, b):
        @pl.when(c == 0)
        def _():
            pltpu.async_copy(etd_hbm.at[idx_v.at[row]], gbufs[b], sems[b])

        @pl.when(c == 1)
        def _():
            pltpu.async_copy(ewae_hbm.at[idx_v.at[row]], gbufs[b], sems[b])

    for b in range(NBUF):
        fire(b, b)

    def group(g, carry):
        for b in range(NBUF):
            row = g * NBUF + b
            pltpu.make_async_copy(
                etd_hbm.at[idx_v.at[row]], gbufs[b], sems[b]).wait()
            gb = gbufs[b]

            def bag_body(bag, bcarry):
                base = bag * HIST
                a0 = gb[base, pl.ds(0, L)]
                a1 = gb[base, pl.ds(16, L)]
                a2 = gb[base, pl.ds(32, L)]
                a3 = gb[base, pl.ds(40, L)]
                for r in range(1, HIST):
                    a0 = a0 + gb[base + r, pl.ds(0, L)]
                    a1 = a1 + gb[base + r, pl.ds(16, L)]
                    a2 = a2 + gb[base + r, pl.ds(32, L)]
                    a3 = a3 + gb[base + r, pl.ds(40, L)]
                for q, a in enumerate((a0, a1, a2, a3)):
                    lr = jnp.where(a >= 0.0, a, a * 0.01)
                    maxacc[q, :] = jnp.maximum(maxacc[q, :], lr)
                return bcarry

            lax.fori_loop(0, BAGS_PER_GATHER, bag_body, 0)

            @pl.when(row + NBUF < N_GATHERS)
            def _():
                fire(row + NBUF, b)
        return carry

    lax.fori_loop(0, GROUPS, group, 0)

    outv[pl.ds(48, L)] = jnp.zeros((L,), jnp.float32)
    outv[pl.ds(0, L)] = maxacc[0, :]
    outv[pl.ds(16, L)] = maxacc[1, :]
    outv[pl.ds(32, L)] = maxacc[2, :]
    outv[pl.ds(40, L)] = maxacc[3, :]
    pltpu.sync_copy(outv, shared.at[s])
    plsc.subcore_barrier()

    @pl.when(s == 0)
    def _():
        pltpu.sync_copy(shared, redv)
        m0 = redv[0, pl.ds(0, L)]
        m1 = redv[0, pl.ds(16, L)]
        m2 = redv[0, pl.ds(32, L)]
        m3 = redv[0, pl.ds(48, L)]
        for t in range(1, NS):
            m0 = jnp.maximum(m0, redv[t, pl.ds(0, L)])
            m1 = jnp.maximum(m1, redv[t, pl.ds(16, L)])
            m2 = jnp.maximum(m2, redv[t, pl.ds(32, L)])
            m3 = jnp.maximum(m3, redv[t, pl.ds(48, L)])
        resv[pl.ds(0, L)] = m0
        resv[pl.ds(16, L)] = m1
        resv[pl.ds(32, L)] = m2
        resv[pl.ds(48, L)] = m3

        @pl.when(c == 0)
        def _():
            pltpu.sync_copy(resv, outr_hbm)

        @pl.when(c == 1)
        def _():
            pltpu.sync_copy(resv, outw_hbm)


REPACK_R = 4096
REPACK_H = REPACK_R // 2
REPACK_G = 25
PACKED_ROWS = REPACK_G * REPACK_H


def _repack_body(a_ref, b_ref, oa_ref, ob_ref):
    z14 = jnp.zeros((REPACK_H, DIMP - DIM), jnp.float32)
    for src_ref, dst_ref in ((a_ref, oa_ref), (b_ref, ob_ref)):
        xt = src_ref[...].T
        dst_ref[:, 0:DIM] = xt[0:REPACK_H, :]
        dst_ref[:, DIM:DIMP] = z14
        dst_ref[:, DIMP:DIMP + DIM] = xt[REPACK_H:REPACK_R, :]
        dst_ref[:, DIMP + DIM:2 * DIMP] = z14


def _repack(E_a, E_b):
    return pl.pallas_call(
        _repack_body,
        grid=(REPACK_G,),
        in_specs=[
            pl.BlockSpec((DIM, REPACK_R), lambda j: (0, j)),
            pl.BlockSpec((DIM, REPACK_R), lambda j: (0, j)),
        ],
        out_specs=[
            pl.BlockSpec((REPACK_H, 2 * DIMP), lambda j: (j, 0)),
            pl.BlockSpec((REPACK_H, 2 * DIMP), lambda j: (j, 0)),
        ],
        out_shape=[
            jax.ShapeDtypeStruct((PACKED_ROWS, 2 * DIMP), jnp.float32),
            jax.ShapeDtypeStruct((PACKED_ROWS, 2 * DIMP), jnp.float32),
        ],
    )(E_a.T, E_b.T)


def _head_body(pr_ref, pw_ref, w_ref, b_ref, y_ref, pred_ref, loss_ref):
    pr = pr_ref[0:1, 0:DIM]
    pw = pw_ref[0:1, 0:DIM]
    path = jnp.concatenate([pr, pw], axis=1)
    w = w_ref[...]
    logits = jnp.sum(w * path, axis=1, keepdims=True).T + b_ref[...]
    m = jnp.max(logits, axis=1, keepdims=True)
    e = jnp.exp(logits - m)
    p = e / jnp.sum(e, axis=1, keepdims=True)
    pred_ref[...] = p
    is_one = y_ref[...] == 1.0
    ii = lax.broadcasted_iota(jnp.int32, (1, CLASS_NUM), 1).astype(jnp.float32)
    cand = jnp.where(is_one, ii, jnp.float32(CLASS_NUM))
    idx_first = jnp.min(cand)
    label = jnp.where(idx_first < CLASS_NUM, idx_first, 0.0)
    sel = (ii == label).astype(jnp.float32)
    pm = jnp.max(p, axis=1, keepdims=True)
    lse = pm + jnp.log(jnp.sum(jnp.exp(p - pm), axis=1, keepdims=True))
    p_label = jnp.sum(p * sel, axis=1, keepdims=True)
    loss_ref[...] = lse - p_label


@jax.jit
def kernel(x_random, x_response, y, E_td, E_wae, w_cat, b_cat):
    def remap(x):
        b = x // REPACK_R
        r = x % REPACK_R
        return 2 * (REPACK_H * b + r % REPACK_H) + r // REPACK_H

    xr = remap(x_random).reshape(NS, N_GATHERS, G_ROWS)
    xs = remap(x_response).reshape(NS, N_GATHERS, G_ROWS)
    etd_p, ewae_p = _repack(E_td, E_wae)
    etd = etd_p.reshape(2 * PACKED_ROWS, DIMP)
    ewae = ewae_p.reshape(2 * PACKED_ROWS, DIMP)

    sc = pl.kernel(
        _sc_body,
        out_type=[
            jax.ShapeDtypeStruct((4 * L,), jnp.float32),
            jax.ShapeDtypeStruct((4 * L,), jnp.float32),
        ],
        mesh=plsc.VectorSubcoreMesh(core_axis_name="c", subcore_axis_name="s"),
        compiler_params=pltpu.CompilerParams(use_tc_tiling_on_sc=False),
        scratch_types=(
            [pltpu.VMEM((N_GATHERS, G_ROWS), jnp.int32)]
            + [pltpu.VMEM((G_ROWS, DIMP), jnp.float32)] * NBUF
            + [
                pltpu.VMEM((4, L), jnp.float32),
                pltpu.VMEM((4 * L,), jnp.float32),
                pltpu.VMEM((NS, 4 * L), jnp.float32),
                pltpu.VMEM((4 * L,), jnp.float32),
                pltpu.VMEM_SHARED((NS, 4 * L), jnp.float32),
            ]
            + [pltpu.SemaphoreType.DMA] * NBUF
        ),
    )
    path_r, path_w = sc(xr, xs, etd, ewae)

    pred2, loss2 = pl.pallas_call(
        _head_body,
        out_shape=[
            jax.ShapeDtypeStruct((1, CLASS_NUM), jnp.float32),
            jax.ShapeDtypeStruct((1, 1), jnp.float32),
        ],
    )(path_r.reshape(1, 4 * L), path_w.reshape(1, 4 * L),
      w_cat, b_cat.reshape(1, CLASS_NUM), y.reshape(1, CLASS_NUM))

    return (pred2[0], loss2[0, 0])
